# baseline (speedup 1.0000x reference)
import jax
import jax.numpy as jnp
from jax.experimental import pallas as pl
from jax.experimental.pallas import tpu as pltpu

HID = 96
PER = 32
NROWS = 30
BLK = 4096
SUBB = BLK // 128
CHUNKS = 4


def _tc_body(cx_ref, cy_ref, cz_ref, xe_ref, ye_ref, ze_ref, w_ref, b_ref,
             *rest):
    out_ref, ttb_ref = rest[-2], rest[-1]

    @pl.when(pl.program_id(0) == 0)
    def _init():
        w = w_ref[...]
        dn = (((1,), (1,)), ((), ()))
        tx = jax.lax.dot_general(xe_ref[...], w[:, 0:32], dn,
                                 preferred_element_type=jnp.float32)
        ty = jax.lax.dot_general(ye_ref[...], w[:, 32:64], dn,
                                 preferred_element_type=jnp.float32)
        tz = jax.lax.dot_general(ze_ref[...], w[:, 64:96], dn,
                                 preferred_element_type=jnp.float32)
        tx = tx + b_ref[...][None, :]
        ttb_ref[...] = jnp.zeros((128, HID), jnp.bfloat16)
        ttb_ref[0:30, :] = tx.astype(jnp.bfloat16)
        ttb_ref[30:31, :] = tx[29:30, :].astype(jnp.bfloat16)
        ttb_ref[32:62, :] = ty.astype(jnp.bfloat16)
        ttb_ref[62:63, :] = ty[29:30, :].astype(jnp.bfloat16)
        ttb_ref[64:94, :] = tz.astype(jnp.bfloat16)
        ttb_ref[94:95, :] = tz[29:30, :].astype(jnp.bfloat16)

    rx = jnp.round(cx_ref[...])
    ry = jnp.round(cy_ref[...]) + 32.0
    rz = jnp.round(cz_ref[...]) + 64.0
    rowc = jax.lax.broadcasted_iota(jnp.int32, (128, 128), 0).astype(
        jnp.float32)
    chunks = []
    for j in range(SUBB):
        m = ((rowc == rx[j:j + 1, :]) | (rowc == ry[j:j + 1, :])
             | (rowc == rz[j:j + 1, :]))
        chunks.append(m)
    ohT = jnp.concatenate(chunks, axis=1).astype(jnp.bfloat16)
    out_ref[...] = jax.lax.dot_general(
        ohT, ttb_ref[...], (((0,), (0,)), ((), ())),
        preferred_element_type=jnp.float32)


def kernel(coords, x_emb, y_emb, z_emb, W, b):
    B, S, _ = coords.shape
    n = B * S
    bc = B // CHUNKS
    nc = n // CHUNKS
    nrc = nc // 128
    blocks_per_chunk = nc // BLK

    cspec = pl.BlockSpec((SUBB, 128), lambda i: (i, 0))
    full_specs = [
        pl.BlockSpec((NROWS, PER), lambda i: (0, 0)),
        pl.BlockSpec((NROWS, PER), lambda i: (0, 0)),
        pl.BlockSpec((NROWS, PER), lambda i: (0, 0)),
        pl.BlockSpec((HID, HID), lambda i: (0, 0)),
        pl.BlockSpec((HID,), lambda i: (0,)),
    ]

    out = None
    tok = None
    for k in range(CHUNKS):
        if tok is None:
            csrc = coords
        else:
            csrc, _ = jax.lax.optimization_barrier((coords, tok))
        csl = csrc[k * bc:(k + 1) * bc]
        cx = csl[..., 0].reshape(nrc, 128)
        cy = csl[..., 1].reshape(nrc, 128)
        cz = csl[..., 2].reshape(nrc, 128)
        tok = jnp.dot(cx[:8], jnp.ones((128, 8), jnp.float32))
        base = k * blocks_per_chunk
        out_spec = pl.BlockSpec((BLK, HID), lambda i, base=base: (i + base, 0))
        in_specs = [cspec, cspec, cspec] + full_specs
        args = [cx, cy, cz, x_emb, y_emb, z_emb, W, b]
        io_aliases = {}
        if out is not None:
            in_specs = in_specs + [pl.BlockSpec(memory_space=pl.ANY)]
            args = args + [out]
            io_aliases = {8: 0}
        out = pl.pallas_call(
            _tc_body,
            grid=(blocks_per_chunk,),
            in_specs=in_specs,
            out_specs=out_spec,
            out_shape=jax.ShapeDtypeStruct((n, HID), jnp.float32),
            scratch_shapes=[pltpu.VMEM((128, HID), jnp.bfloat16)],
            input_output_aliases=io_aliases,
        )(*args)
    return out.reshape(B, S, HID)


# speedup vs baseline: 1.1211x; 1.1211x over previous
import jax
import jax.numpy as jnp
from jax.experimental import pallas as pl
from jax.experimental.pallas import tpu as pltpu

HID = 96
PER = 32
NROWS = 30
BLK = 8192
SUBB = BLK // 128


def _tc_body(cx_ref, cy_ref, cz_ref, xe_ref, ye_ref, ze_ref, w_ref, b_ref,
             out_ref, ttb_ref):
    @pl.when(pl.program_id(0) == 0)
    def _init():
        w = w_ref[...]
        dn = (((1,), (1,)), ((), ()))
        tx = jax.lax.dot_general(xe_ref[...], w[:, 0:32], dn,
                                 preferred_element_type=jnp.float32)
        ty = jax.lax.dot_general(ye_ref[...], w[:, 32:64], dn,
                                 preferred_element_type=jnp.float32)
        tz = jax.lax.dot_general(ze_ref[...], w[:, 64:96], dn,
                                 preferred_element_type=jnp.float32)
        tx = tx + b_ref[...][None, :]
        ttb_ref[...] = jnp.zeros((128, HID), jnp.bfloat16)
        ttb_ref[0:30, :] = tx.astype(jnp.bfloat16)
        ttb_ref[30:31, :] = tx[29:30, :].astype(jnp.bfloat16)
        ttb_ref[32:62, :] = ty.astype(jnp.bfloat16)
        ttb_ref[62:63, :] = ty[29:30, :].astype(jnp.bfloat16)
        ttb_ref[64:94, :] = tz.astype(jnp.bfloat16)
        ttb_ref[94:95, :] = tz[29:30, :].astype(jnp.bfloat16)

    rx = jnp.round(cx_ref[...])
    ry = jnp.round(cy_ref[...]) + 32.0
    rz = jnp.round(cz_ref[...]) + 64.0
    rowc = jax.lax.broadcasted_iota(jnp.int32, (128, 128), 0).astype(
        jnp.float32)
    chunks = []
    for j in range(SUBB):
        m = ((rowc == rx[j:j + 1, :]) | (rowc == ry[j:j + 1, :])
             | (rowc == rz[j:j + 1, :]))
        chunks.append(m)
    ohT = jnp.concatenate(chunks, axis=1).astype(jnp.bfloat16)
    out_ref[...] = jax.lax.dot_general(
        ohT, ttb_ref[...], (((0,), (0,)), ((), ())),
        preferred_element_type=jnp.float32)


def kernel(coords, x_emb, y_emb, z_emb, W, b):
    B, S, _ = coords.shape
    n = B * S
    nr = n // 128
    cx = coords[..., 0].reshape(nr, 128)
    cy = coords[..., 1].reshape(nr, 128)
    cz = coords[..., 2].reshape(nr, 128)
    cspec = pl.BlockSpec((SUBB, 128), lambda i: (i, 0))
    out = pl.pallas_call(
        _tc_body,
        grid=(n // BLK,),
        in_specs=[
            cspec, cspec, cspec,
            pl.BlockSpec((NROWS, PER), lambda i: (0, 0)),
            pl.BlockSpec((NROWS, PER), lambda i: (0, 0)),
            pl.BlockSpec((NROWS, PER), lambda i: (0, 0)),
            pl.BlockSpec((HID, HID), lambda i: (0, 0)),
            pl.BlockSpec((HID,), lambda i: (0,)),
        ],
        out_specs=pl.BlockSpec((BLK, HID), lambda i: (i, 0)),
        out_shape=jax.ShapeDtypeStruct((n, HID), jnp.float32),
        scratch_shapes=[pltpu.VMEM((128, HID), jnp.bfloat16)],
    )(cx, cy, cz, x_emb, y_emb, z_emb, W, b)
    return out.reshape(B, S, HID)


# speedup vs baseline: 1.1730x; 1.0463x over previous
import jax
import jax.numpy as jnp
from jax.experimental import pallas as pl
from jax.experimental.pallas import tpu as pltpu

HID = 96
PER = 32
NROWS = 30
BLK = 16384
SUBB = BLK // 128


def _tc_body(cx_ref, cy_ref, cz_ref, xe_ref, ye_ref, ze_ref, w_ref, b_ref,
             out_ref, ttb_ref):
    @pl.when(pl.program_id(0) == 0)
    def _init():
        w = w_ref[...]
        dn = (((1,), (1,)), ((), ()))
        tx = jax.lax.dot_general(xe_ref[...], w[:, 0:32], dn,
                                 preferred_element_type=jnp.float32)
        ty = jax.lax.dot_general(ye_ref[...], w[:, 32:64], dn,
                                 preferred_element_type=jnp.float32)
        tz = jax.lax.dot_general(ze_ref[...], w[:, 64:96], dn,
                                 preferred_element_type=jnp.float32)
        tx = tx + b_ref[...][None, :]
        ttb_ref[...] = jnp.zeros((128, HID), jnp.bfloat16)
        ttb_ref[0:30, :] = tx.astype(jnp.bfloat16)
        ttb_ref[30:31, :] = tx[29:30, :].astype(jnp.bfloat16)
        ttb_ref[32:62, :] = ty.astype(jnp.bfloat16)
        ttb_ref[62:63, :] = ty[29:30, :].astype(jnp.bfloat16)
        ttb_ref[64:94, :] = tz.astype(jnp.bfloat16)
        ttb_ref[94:95, :] = tz[29:30, :].astype(jnp.bfloat16)

    rx = jnp.round(cx_ref[...])
    ry = jnp.round(cy_ref[...]) + 32.0
    rz = jnp.round(cz_ref[...]) + 64.0
    rowc = jax.lax.broadcasted_iota(jnp.int32, (128, 128), 0).astype(
        jnp.float32)
    chunks = []
    for j in range(SUBB):
        m = ((rowc == rx[j:j + 1, :]) | (rowc == ry[j:j + 1, :])
             | (rowc == rz[j:j + 1, :]))
        chunks.append(m)
    ohT = jnp.concatenate(chunks, axis=1).astype(jnp.bfloat16)
    out_ref[...] = jax.lax.dot_general(
        ohT, ttb_ref[...], (((0,), (0,)), ((), ())),
        preferred_element_type=jnp.float32)


def kernel(coords, x_emb, y_emb, z_emb, W, b):
    B, S, _ = coords.shape
    n = B * S
    nr = n // 128
    cx = coords[..., 0].reshape(nr, 128)
    cy = coords[..., 1].reshape(nr, 128)
    cz = coords[..., 2].reshape(nr, 128)
    cspec = pl.BlockSpec((SUBB, 128), lambda i: (i, 0))
    out = pl.pallas_call(
        _tc_body,
        grid=(n // BLK,),
        in_specs=[
            cspec, cspec, cspec,
            pl.BlockSpec((NROWS, PER), lambda i: (0, 0)),
            pl.BlockSpec((NROWS, PER), lambda i: (0, 0)),
            pl.BlockSpec((NROWS, PER), lambda i: (0, 0)),
            pl.BlockSpec((HID, HID), lambda i: (0, 0)),
            pl.BlockSpec((HID,), lambda i: (0,)),
        ],
        out_specs=pl.BlockSpec((BLK, HID), lambda i: (i, 0)),
        out_shape=jax.ShapeDtypeStruct((n, HID), jnp.float32),
        scratch_shapes=[pltpu.VMEM((128, HID), jnp.bfloat16)],
    )(cx, cy, cz, x_emb, y_emb, z_emb, W, b)
    return out.reshape(B, S, HID)


# speedup vs baseline: 1.1827x; 1.0083x over previous
import jax
import jax.numpy as jnp
from jax.experimental import pallas as pl
from jax.experimental.pallas import tpu as pltpu

HID = 96
PER = 32
NROWS = 30
BLK = 32768
SUBB = BLK // 128


def _tc_body(cx_ref, cy_ref, cz_ref, xe_ref, ye_ref, ze_ref, w_ref, b_ref,
             out_ref, ttb_ref):
    @pl.when(pl.program_id(0) == 0)
    def _init():
        w = w_ref[...]
        dn = (((1,), (1,)), ((), ()))
        tx = jax.lax.dot_general(xe_ref[...], w[:, 0:32], dn,
                                 preferred_element_type=jnp.float32)
        ty = jax.lax.dot_general(ye_ref[...], w[:, 32:64], dn,
                                 preferred_element_type=jnp.float32)
        tz = jax.lax.dot_general(ze_ref[...], w[:, 64:96], dn,
                                 preferred_element_type=jnp.float32)
        tx = tx + b_ref[...][None, :]
        ttb_ref[...] = jnp.zeros((128, HID), jnp.bfloat16)
        ttb_ref[0:30, :] = tx.astype(jnp.bfloat16)
        ttb_ref[30:31, :] = tx[29:30, :].astype(jnp.bfloat16)
        ttb_ref[32:62, :] = ty.astype(jnp.bfloat16)
        ttb_ref[62:63, :] = ty[29:30, :].astype(jnp.bfloat16)
        ttb_ref[64:94, :] = tz.astype(jnp.bfloat16)
        ttb_ref[94:95, :] = tz[29:30, :].astype(jnp.bfloat16)

    rx = jnp.round(cx_ref[...])
    ry = jnp.round(cy_ref[...]) + 32.0
    rz = jnp.round(cz_ref[...]) + 64.0
    rowc = jax.lax.broadcasted_iota(jnp.int32, (128, 128), 0).astype(
        jnp.float32)
    chunks = []
    for j in range(SUBB):
        m = ((rowc == rx[j:j + 1, :]) | (rowc == ry[j:j + 1, :])
             | (rowc == rz[j:j + 1, :]))
        chunks.append(m)
    ohT = jnp.concatenate(chunks, axis=1).astype(jnp.bfloat16)
    out_ref[...] = jax.lax.dot_general(
        ohT, ttb_ref[...], (((0,), (0,)), ((), ())),
        preferred_element_type=jnp.float32)


def kernel(coords, x_emb, y_emb, z_emb, W, b):
    B, S, _ = coords.shape
    n = B * S
    nr = n // 128
    cx = coords[..., 0].reshape(nr, 128)
    cy = coords[..., 1].reshape(nr, 128)
    cz = coords[..., 2].reshape(nr, 128)
    cspec = pl.BlockSpec((SUBB, 128), lambda i: (i, 0))
    out = pl.pallas_call(
        _tc_body,
        grid=(n // BLK,),
        in_specs=[
            cspec, cspec, cspec,
            pl.BlockSpec((NROWS, PER), lambda i: (0, 0)),
            pl.BlockSpec((NROWS, PER), lambda i: (0, 0)),
            pl.BlockSpec((NROWS, PER), lambda i: (0, 0)),
            pl.BlockSpec((HID, HID), lambda i: (0, 0)),
            pl.BlockSpec((HID,), lambda i: (0,)),
        ],
        out_specs=pl.BlockSpec((BLK, HID), lambda i: (i, 0)),
        out_shape=jax.ShapeDtypeStruct((n, HID), jnp.float32),
        scratch_shapes=[pltpu.VMEM((128, HID), jnp.bfloat16)],
    )(cx, cy, cz, x_emb, y_emb, z_emb, W, b)
    return out.reshape(B, S, HID)
